# speedup vs baseline: 1.1164x; 1.1164x over previous
import functools

import jax
import jax.numpy as jnp
from jax import lax
from jax.experimental import pallas as pl
from jax.experimental.pallas import tpu as pltpu
from jax.experimental.pallas import tpu_sc as plsc

NUM_NODES = 100000
H_DIM = 256
CHUNK = 80
NUM_CHUNKS = NUM_NODES // CHUNK
NC = 2
NS = 16
NW = NC * NS
CHUNKS_PER_W = (NUM_CHUNKS + NW - 1) // NW

_mesh = plsc.VectorSubcoreMesh(core_axis_name="c", subcore_axis_name="s")


@functools.partial(
    pl.kernel,
    mesh=_mesh,
    out_type=jax.ShapeDtypeStruct((NUM_NODES, H_DIM), jnp.float32),
    scratch_types=[
        pltpu.VMEM((CHUNK,), jnp.int32),
        pltpu.VMEM((CHUNK, H_DIM), jnp.float32),
        pltpu.SemaphoreType.DMA,
    ],
)
def _gather_kernel(idx_hbm, table_hbm, out_hbm, idx_v, rows_v, sem):
    w = lax.axis_index("s") * NC + lax.axis_index("c")

    def body(k, carry):
        c = w + NW * k

        @pl.when(c < NUM_CHUNKS)
        def _():
            base = c * CHUNK
            pltpu.sync_copy(idx_hbm.at[pl.ds(base, CHUNK)], idx_v)
            pltpu.async_copy(table_hbm.at[idx_v], rows_v, sem).wait()
            pltpu.sync_copy(rows_v, out_hbm.at[pl.ds(base, CHUNK)])

        return carry

    lax.fori_loop(0, CHUNKS_PER_W, body, 0)


def kernel(node_id, weight, incidence_in, incidence_out, table):
    node_id = jnp.squeeze(node_id)
    h = _gather_kernel(node_id, table)
    return (weight, h)


# speedup vs baseline: 1.6035x; 1.4363x over previous
import functools

import jax
import jax.numpy as jnp
from jax import lax
from jax.experimental import pallas as pl
from jax.experimental.pallas import tpu as pltpu
from jax.experimental.pallas import tpu_sc as plsc

NUM_NODES = 100000
H_DIM = 256
CHUNK = 80
NUM_CHUNKS = NUM_NODES // CHUNK
NC = 2
NS = 16
NW = NC * NS
BASE_CHUNKS = NUM_CHUNKS // NW
EXTRA_W = NUM_CHUNKS - BASE_CHUNKS * NW
MAX_CHUNKS = BASE_CHUNKS + 1
NBUF = 4
LOOKAHEAD = 2

_mesh = plsc.VectorSubcoreMesh(core_axis_name="c", subcore_axis_name="s")


@functools.partial(
    pl.kernel,
    mesh=_mesh,
    out_type=jax.ShapeDtypeStruct((NUM_NODES, H_DIM), jnp.float32),
    scratch_types=[pltpu.VMEM((MAX_CHUNKS * CHUNK,), jnp.int32)]
    + [pltpu.VMEM((CHUNK, H_DIM), jnp.float32) for _ in range(NBUF)]
    + [pltpu.SemaphoreType.DMA for _ in range(2 * NBUF)],
)
def _gather_kernel(idx_hbm, table_hbm, out_hbm, idx_all, r0, r1, r2, r3,
                   g0, g1, g2, g3, s0, s1, s2, s3):
    rows = [r0, r1, r2, r3]
    gsem = [g0, g1, g2, g3]
    wsem = [s0, s1, s2, s3]

    w = lax.axis_index("s") * NC + lax.axis_index("c")
    lo = BASE_CHUNKS * w + jnp.minimum(w, EXTRA_W)
    has_extra = w < EXTRA_W

    base_el = lo * CHUNK
    n_base = BASE_CHUNKS * CHUNK
    pltpu.sync_copy(idx_hbm.at[pl.ds(base_el, n_base)],
                    idx_all.at[pl.ds(0, n_base)])

    @pl.when(has_extra)
    def _():
        pltpu.sync_copy(idx_hbm.at[pl.ds(base_el + n_base, CHUNK)],
                        idx_all.at[pl.ds(n_base, CHUNK)])

    def gather_desc(j, b):
        idx_slice = idx_all.at[pl.ds(j * CHUNK, CHUNK)]
        return pltpu.make_async_copy(table_hbm.at[idx_slice], rows[b],
                                     gsem[b])

    def write_desc(j, b):
        dst = out_hbm.at[pl.ds((lo + j) * CHUNK, CHUNK)]
        return pltpu.make_async_copy(rows[b], dst, wsem[b])

    def guarded(j, fn):
        if j < 0 or j >= MAX_CHUNKS:
            return
        if j < BASE_CHUNKS:
            fn()
        else:
            pl.when(has_extra)(fn)

    for j in range(LOOKAHEAD):
        guarded(j, lambda j=j: gather_desc(j, j % NBUF).start())

    for j in range(MAX_CHUNKS):
        b = j % NBUF
        nj = j + LOOKAHEAD
        guarded(nj - NBUF, lambda j=nj - NBUF: write_desc(j, j % NBUF).wait())
        guarded(nj, lambda j=nj: gather_desc(j, j % NBUF).start())
        guarded(j, lambda j=j, b=b: gather_desc(j, b).wait())
        guarded(j, lambda j=j, b=b: write_desc(j, b).start())

    for j in range(MAX_CHUNKS - NBUF + LOOKAHEAD, MAX_CHUNKS):
        guarded(j, lambda j=j: write_desc(j, j % NBUF).wait())


def kernel(node_id, weight, incidence_in, incidence_out, table):
    node_id = jnp.squeeze(node_id)
    h = _gather_kernel(node_id, table)
    return (weight, h)


# speedup vs baseline: 1.6208x; 1.0108x over previous
import functools

import jax
import jax.numpy as jnp
from jax import lax
from jax.experimental import pallas as pl
from jax.experimental.pallas import tpu as pltpu
from jax.experimental.pallas import tpu_sc as plsc

NUM_NODES = 100000
H_DIM = 256
CHUNK = 80
NUM_CHUNKS = NUM_NODES // CHUNK
NC = 2
NS = 16
NW = NC * NS
BASE_CHUNKS = NUM_CHUNKS // NW
EXTRA_W = NUM_CHUNKS - BASE_CHUNKS * NW
MAX_CHUNKS = BASE_CHUNKS + 1
NBUF = 6
LOOKAHEAD = 3

_mesh = plsc.VectorSubcoreMesh(core_axis_name="c", subcore_axis_name="s")


@functools.partial(
    pl.kernel,
    mesh=_mesh,
    out_type=jax.ShapeDtypeStruct((NUM_NODES, H_DIM), jnp.float32),
    scratch_types=[pltpu.VMEM((MAX_CHUNKS * CHUNK,), jnp.int32)]
    + [pltpu.VMEM((CHUNK, H_DIM), jnp.float32) for _ in range(NBUF)]
    + [pltpu.SemaphoreType.DMA for _ in range(2 * NBUF)],
)
def _gather_kernel(idx_hbm, table_hbm, out_hbm, idx_all, *scratch):
    rows = list(scratch[:NBUF])
    gsem = list(scratch[NBUF:2 * NBUF])
    wsem = list(scratch[2 * NBUF:])

    w = lax.axis_index("s") * NC + lax.axis_index("c")
    lo = BASE_CHUNKS * w + jnp.minimum(w, EXTRA_W)
    has_extra = w < EXTRA_W

    base_el = lo * CHUNK
    n_base = BASE_CHUNKS * CHUNK
    pltpu.sync_copy(idx_hbm.at[pl.ds(base_el, n_base)],
                    idx_all.at[pl.ds(0, n_base)])

    @pl.when(has_extra)
    def _():
        pltpu.sync_copy(idx_hbm.at[pl.ds(base_el + n_base, CHUNK)],
                        idx_all.at[pl.ds(n_base, CHUNK)])

    def gather_desc(j, b):
        idx_slice = idx_all.at[pl.ds(j * CHUNK, CHUNK)]
        return pltpu.make_async_copy(table_hbm.at[idx_slice], rows[b],
                                     gsem[b])

    def write_desc(j, b):
        dst = out_hbm.at[pl.ds((lo + j) * CHUNK, CHUNK)]
        return pltpu.make_async_copy(rows[b], dst, wsem[b])

    def guarded(j, fn):
        if j < 0 or j >= MAX_CHUNKS:
            return
        if j < BASE_CHUNKS:
            fn()
        else:
            pl.when(has_extra)(fn)

    for j in range(LOOKAHEAD):
        guarded(j, lambda j=j: gather_desc(j, j % NBUF).start())

    for j in range(MAX_CHUNKS):
        b = j % NBUF
        nj = j + LOOKAHEAD
        guarded(nj - NBUF, lambda j=nj - NBUF: write_desc(j, j % NBUF).wait())
        guarded(nj, lambda j=nj: gather_desc(j, j % NBUF).start())
        guarded(j, lambda j=j, b=b: gather_desc(j, b).wait())
        guarded(j, lambda j=j, b=b: write_desc(j, b).start())

    for j in range(MAX_CHUNKS - NBUF + LOOKAHEAD, MAX_CHUNKS):
        guarded(j, lambda j=j: write_desc(j, j % NBUF).wait())


def kernel(node_id, weight, incidence_in, incidence_out, table):
    node_id = jnp.squeeze(node_id)
    h = _gather_kernel(node_id, table)
    return (weight, h)


# speedup vs baseline: 1.6727x; 1.0320x over previous
import functools

import jax
import jax.numpy as jnp
from jax import lax
from jax.experimental import pallas as pl
from jax.experimental.pallas import tpu as pltpu
from jax.experimental.pallas import tpu_sc as plsc

NUM_NODES = 100000
H_DIM = 256
CHUNK = 80
NUM_CHUNKS = NUM_NODES // CHUNK
NC = 2
NS = 16
NW = NC * NS
BASE_CHUNKS = NUM_CHUNKS // NW
EXTRA_W = NUM_CHUNKS - BASE_CHUNKS * NW
MAX_CHUNKS = BASE_CHUNKS + 1
NBUF = 6
LOOKAHEAD = 3
NITER = -(-MAX_CHUNKS // NBUF)

_mesh = plsc.VectorSubcoreMesh(core_axis_name="c", subcore_axis_name="s")


@functools.partial(
    pl.kernel,
    mesh=_mesh,
    out_type=jax.ShapeDtypeStruct((NUM_NODES, H_DIM), jnp.float32),
    scratch_types=[pltpu.VMEM((MAX_CHUNKS * CHUNK,), jnp.int32)]
    + [pltpu.VMEM((CHUNK, H_DIM), jnp.float32) for _ in range(NBUF)]
    + [pltpu.SemaphoreType.DMA for _ in range(2 * NBUF)],
)
def _gather_kernel(idx_hbm, table_hbm, out_hbm, idx_all, *scratch):
    rows = list(scratch[:NBUF])
    gsem = list(scratch[NBUF:2 * NBUF])
    wsem = list(scratch[2 * NBUF:])

    w = lax.axis_index("s") * NC + lax.axis_index("c")
    lo = BASE_CHUNKS * w + jnp.minimum(w, EXTRA_W)
    n_w = BASE_CHUNKS + jnp.where(w < EXTRA_W, 1, 0)

    base_el = lo * CHUNK
    n_base = BASE_CHUNKS * CHUNK
    pltpu.sync_copy(idx_hbm.at[pl.ds(base_el, n_base)],
                    idx_all.at[pl.ds(0, n_base)])

    @pl.when(w < EXTRA_W)
    def _():
        pltpu.sync_copy(idx_hbm.at[pl.ds(base_el + n_base, CHUNK)],
                        idx_all.at[pl.ds(n_base, CHUNK)])

    def gather_desc(j, b):
        off = pl.multiple_of(j * CHUNK, CHUNK)
        idx_slice = idx_all.at[pl.ds(off, CHUNK)]
        return pltpu.make_async_copy(table_hbm.at[idx_slice], rows[b],
                                     gsem[b])

    def write_desc(j, b):
        dst = out_hbm.at[pl.ds((lo + j) * CHUNK, CHUNK)]
        return pltpu.make_async_copy(rows[b], dst, wsem[b])

    for b in range(NBUF):
        gather_desc(b, b).start()

    def ring(it, carry):
        for b in range(NBUF):
            j = it * NBUF + b
            jn = j + LOOKAHEAD
            bn = (b + LOOKAHEAD) % NBUF

            @pl.when((jn >= NBUF) & (jn < n_w))
            def _(j=jn, b=bn):
                write_desc(j - NBUF, b).wait()
                gather_desc(j, b).start()

            @pl.when(j < n_w)
            def _(j=j, b=b):
                gather_desc(j, b).wait()
                write_desc(j, b).start()

        return carry

    lax.fori_loop(0, NITER, ring, 0)

    for b in range(NBUF):
        write_desc(b, b).wait()


def kernel(node_id, weight, incidence_in, incidence_out, table):
    node_id = jnp.squeeze(node_id)
    h = _gather_kernel(node_id, table)
    return (weight, h)
